# baseline (speedup 1.0000x reference)
import functools

import jax
import jax.numpy as jnp
from jax.experimental import pallas as pl
from jax.experimental.pallas import tpu as pltpu


@functools.lru_cache(maxsize=None)
def _build(seq_len: int, nhid: int):
    chunk = 512
    assert seq_len % chunk == 0
    n = seq_len // chunk

    def body(pe_ref, out_ref, buf, in_sems, out_sems):
        def cp_in(i):
            return pltpu.make_async_copy(
                pe_ref.at[pl.ds(i * chunk, chunk)], buf.at[i], in_sems.at[i])

        def cp_out(i):
            return pltpu.make_async_copy(
                buf.at[i], out_ref.at[pl.ds(i * chunk, chunk)], out_sems.at[i])

        for i in range(n):
            cp_in(i).start()
        for i in range(n):
            cp_in(i).wait()
            cp_out(i).start()
        for i in range(n):
            cp_out(i).wait()

    return pl.pallas_call(
        body,
        in_specs=[pl.BlockSpec(memory_space=pl.ANY)],
        out_specs=pl.BlockSpec(memory_space=pl.ANY),
        out_shape=jax.ShapeDtypeStruct((seq_len, nhid), jnp.float32),
        scratch_shapes=[
            pltpu.VMEM((n, chunk, nhid), jnp.float32),
            pltpu.SemaphoreType.DMA((n,)),
            pltpu.SemaphoreType.DMA((n,)),
        ],
    )


def kernel(x, pe):
    seq_len = x.shape[1]
    nhid = pe.shape[2]
    out2d = _build(seq_len, nhid)(pe.reshape(pe.shape[1], nhid))
    return out2d.reshape(1, seq_len, nhid)


# speedup vs baseline: 1.0629x; 1.0629x over previous
import functools

import jax
import jax.numpy as jnp
from jax.experimental import pallas as pl
from jax.experimental.pallas import tpu as pltpu


@functools.lru_cache(maxsize=None)
def _build(seq_len: int, nhid: int):
    frac = (1, 7, 7, 1)
    unit = seq_len // sum(frac)
    chunks = [f * unit for f in frac]
    offs = [sum(chunks[:i]) for i in range(len(chunks))]
    n = len(chunks)

    def body(pe_ref, out_ref, buf, in_sems, out_sems):
        def cp_in(i):
            return pltpu.make_async_copy(
                pe_ref.at[pl.ds(offs[i], chunks[i])],
                buf.at[pl.ds(offs[i], chunks[i])], in_sems.at[i])

        def cp_out(i):
            return pltpu.make_async_copy(
                buf.at[pl.ds(offs[i], chunks[i])],
                out_ref.at[pl.ds(offs[i], chunks[i])], out_sems.at[i])

        for i in range(n):
            cp_in(i).start()
        for i in range(n):
            cp_in(i).wait()
            cp_out(i).start()
        for i in range(n):
            cp_out(i).wait()

    return pl.pallas_call(
        body,
        in_specs=[pl.BlockSpec(memory_space=pl.ANY)],
        out_specs=pl.BlockSpec(memory_space=pl.ANY),
        out_shape=jax.ShapeDtypeStruct((seq_len, nhid), jnp.float32),
        scratch_shapes=[
            pltpu.VMEM((seq_len, nhid), jnp.float32),
            pltpu.SemaphoreType.DMA((n,)),
            pltpu.SemaphoreType.DMA((n,)),
        ],
    )


def kernel(x, pe):
    seq_len = x.shape[1]
    nhid = pe.shape[2]
    out2d = _build(seq_len, nhid)(pe.reshape(pe.shape[1], nhid))
    return out2d.reshape(1, seq_len, nhid)
